# baseline (speedup 1.0000x reference)
import functools

import jax
import jax.numpy as jnp
from jax import lax
from jax.experimental import pallas as pl
from jax.experimental.pallas import tpu as pltpu
from jax.experimental.pallas import tpu_sc as plsc

B = 16384
D = 128
LANES = 16
NW = 32
NSUB = 16
ROWS_PER_W = B // NW
CHUNK = 64
NCHUNK = ROWS_PER_W // CHUNK
TROWS = 1000
SPLIT_HI = 64
SPLIT_LO = 40


def _body(x_hbm, lab_hbm, mu_hbm, la_hbm, eps_hbm, out_hbm,
          idx0, idx1, mu0, mu1, al0, al1, x0, x1, e0, e1, o0, o1,
          stg_s, stg_d, mu_tab, al_tab,
          sm0, sm1, sa0, sa1, sx0, sx1, se0, se1, sem_out0, sem_out1):
    idx_v = (idx0, idx1)
    mu_v = (mu0, mu1)
    al_v = (al0, al1)
    x_v = (x0, x1)
    eps_v = (e0, e1)
    out_v = (o0, o1)
    sem_m = (sm0, sm1)
    sem_a = (sa0, sa1)
    sem_x = (sx0, sx1)
    sem_e = (se0, se1)
    sem_out = (sem_out0, sem_out1)

    cid = lax.axis_index("c")
    sid = lax.axis_index("s")
    wid = sid * 2 + cid
    wbase = wid * ROWS_PER_W

    pre = []
    for p in range(2):
        base = wbase + p * CHUNK
        pltpu.sync_copy(lab_hbm.at[pl.ds(base, CHUNK)], idx_v[p])
        pre.append(pltpu.async_copy(x_hbm.at[pl.ds(base, CHUNK)], x_v[p], sem_x[p]))
        pre.append(pltpu.async_copy(eps_hbm.at[pl.ds(base, CHUNK)], eps_v[p], sem_e[p]))

    r0 = sid * SPLIT_HI
    nrows = jnp.where(sid < NSUB - 1, SPLIT_HI, SPLIT_LO)

    @pl.when(sid < NSUB - 1)
    def _():
        pltpu.sync_copy(mu_hbm.at[pl.ds(r0, SPLIT_HI)],
                        mu_tab.at[pl.ds(r0, SPLIT_HI)])
        pltpu.sync_copy(la_hbm.at[pl.ds(r0, SPLIT_HI)],
                        stg_s.at[pl.ds(0, SPLIT_HI)])

    @pl.when(sid == NSUB - 1)
    def _():
        pltpu.sync_copy(mu_hbm.at[pl.ds(r0, SPLIT_LO)],
                        mu_tab.at[pl.ds(r0, SPLIT_LO)])
        pltpu.sync_copy(la_hbm.at[pl.ds(r0, SPLIT_LO)],
                        stg_s.at[pl.ds(0, SPLIT_LO)])

    def exp_row(r, carry):
        for j in range(D // LANES):
            sl = pl.ds(j * LANES, LANES)
            stg_d[r, sl] = jnp.exp(stg_s[r, sl])
        return carry

    lax.fori_loop(0, nrows, exp_row, 0)

    @pl.when(sid < NSUB - 1)
    def _():
        pltpu.sync_copy(stg_d.at[pl.ds(0, SPLIT_HI)],
                        al_tab.at[pl.ds(r0, SPLIT_HI)])

    @pl.when(sid == NSUB - 1)
    def _():
        pltpu.sync_copy(stg_d.at[pl.ds(0, SPLIT_LO)],
                        al_tab.at[pl.ds(r0, SPLIT_LO)])

    plsc.subcore_barrier()

    copies = [list(pre[0:2]), list(pre[2:4])]
    outcp = [None, None]

    def start_gather(ch):
        p = ch % 2
        copies[p].append(pltpu.async_copy(mu_tab.at[idx_v[p]], mu_v[p], sem_m[p]))
        copies[p].append(pltpu.async_copy(al_tab.at[idx_v[p]], al_v[p], sem_a[p]))

    def start_all(ch):
        p = ch % 2
        base = wbase + ch * CHUNK
        pltpu.sync_copy(lab_hbm.at[pl.ds(base, CHUNK)], idx_v[p])
        copies[p] = [
            pltpu.async_copy(x_hbm.at[pl.ds(base, CHUNK)], x_v[p], sem_x[p]),
            pltpu.async_copy(eps_hbm.at[pl.ds(base, CHUNK)], eps_v[p], sem_e[p]),
        ]
        start_gather(ch)

    start_gather(0)
    start_gather(1)
    for ch in range(NCHUNK):
        p = ch % 2
        for c in copies[p]:
            c.wait()
        if outcp[p] is not None:
            outcp[p].wait()

        xv, ev, av, mv, ov = x_v[p], eps_v[p], al_v[p], mu_v[p], out_v[p]

        def row_body(r, carry):
            for j in range(D // LANES):
                sl = pl.ds(j * LANES, LANES)
                ov[r, sl] = xv[r, sl] * (ev[r, sl] * av[r, sl] + mv[r, sl])
            return carry

        lax.fori_loop(0, CHUNK, row_body, 0)
        outcp[p] = pltpu.async_copy(
            out_v[p], out_hbm.at[pl.ds(wbase + ch * CHUNK, CHUNK)], sem_out[p])
        if ch + 2 < NCHUNK:
            start_all(ch + 2)
    for p in range(2):
        if outcp[p] is not None:
            outcp[p].wait()


@jax.jit
def _gauss_dropout_sc(x, labels, muy, log_alpha, epsilon):
    mesh = plsc.VectorSubcoreMesh(core_axis_name="c", subcore_axis_name="s")
    buf = lambda: pltpu.VMEM((CHUNK, D), jnp.float32)
    kfn = functools.partial(
        pl.kernel,
        mesh=mesh,
        out_type=jax.ShapeDtypeStruct((B, D), jnp.float32),
        scratch_types=[
            pltpu.VMEM((CHUNK,), jnp.int32), pltpu.VMEM((CHUNK,), jnp.int32),
            buf(), buf(), buf(), buf(), buf(), buf(), buf(), buf(), buf(), buf(),
            pltpu.VMEM((SPLIT_HI, D), jnp.float32),
            pltpu.VMEM((SPLIT_HI, D), jnp.float32),
            pltpu.VMEM_SHARED((TROWS, D), jnp.float32),
            pltpu.VMEM_SHARED((TROWS, D), jnp.float32),
            pltpu.SemaphoreType.DMA, pltpu.SemaphoreType.DMA,
            pltpu.SemaphoreType.DMA, pltpu.SemaphoreType.DMA,
            pltpu.SemaphoreType.DMA, pltpu.SemaphoreType.DMA,
            pltpu.SemaphoreType.DMA, pltpu.SemaphoreType.DMA,
            pltpu.SemaphoreType.DMA, pltpu.SemaphoreType.DMA,
        ],
    )(_body)
    return kfn(x, labels, muy, log_alpha, epsilon)


def kernel(x, task_labels, muy, log_alpha, epsilon):
    labels = task_labels.astype(jnp.int32)
    return _gauss_dropout_sc(x, labels, muy, log_alpha, epsilon)
